# baseline (device time: 47400 ns/iter reference)
import jax
import jax.numpy as jnp
from jax import lax
from jax.experimental import pallas as pl
from jax.experimental.pallas import tpu as pltpu

N_DEV = 8
B_PER = 2
SQ = 128
SKV = 128
HQ_PER = 4
DH = 64
D_MODEL = 512
QKD = HQ_PER * DH

_MASKS = (1, 2, 4, 3, 5, 6, 7)


def _code(p):
    return p ^ ((p >> 1) & 1)


def kernel(x, Wq, K_ext, V_ext, Wo):
    my = lax.axis_index("i")
    bf16 = jnp.bfloat16

    x2d = x.reshape(B_PER * SQ, D_MODEL).astype(bf16)
    wq = Wq.astype(bf16)
    wo = Wo.astype(bf16)

    def prep(ext):
        eb = lax.dynamic_slice_in_dim(ext, B_PER * my, B_PER, axis=0)
        eb = eb.astype(bf16).reshape(B_PER, SKV, N_DEV, HQ_PER, DH)
        masks = jnp.asarray((0,) + _MASKS)
        idx = _code(_code(my) ^ masks)
        eb = jnp.take(eb, idx, axis=2)
        return jnp.transpose(eb, (2, 0, 3, 1, 4))

    k_arr = prep(K_ext)
    v_arr = prep(V_ext)

    def body(x_ref, wq_ref, wo_ref, k_ref, v_ref, out_ref,
             gwq, gwo, ctx_scratch,
             send_wq, recv_wq, send_wo, recv_wo):
        my_code = _code(lax.axis_index("i"))

        barrier = pltpu.get_barrier_semaphore()
        for m in _MASKS:
            peer = _code(my_code ^ m)
            pl.semaphore_signal(barrier, inc=1, device_id=(peer,),
                                device_id_type=pl.DeviceIdType.MESH)
        pl.semaphore_wait(barrier, N_DEV - 1)

        rds = []
        for s, m in enumerate(_MASKS):
            peer = _code(my_code ^ m)
            rd_wq = pltpu.make_async_remote_copy(
                src_ref=wq_ref, dst_ref=gwq.at[s],
                send_sem=send_wq.at[s], recv_sem=recv_wq.at[s],
                device_id=(peer,), device_id_type=pl.DeviceIdType.MESH)
            rd_wo = pltpu.make_async_remote_copy(
                src_ref=wo_ref, dst_ref=gwo.at[s],
                send_sem=send_wo.at[s], recv_sem=recv_wo.at[s],
                device_id=(peer,), device_id_type=pl.DeviceIdType.MESH)
            rd_wq.start()
            rd_wo.start()
            rds.append((rd_wq, rd_wo))

        x_val = x_ref[...]

        def contrib(t, wq_val, wo_val, is_first=False):
            q2 = lax.dot(x_val, wq_val,
                         preferred_element_type=jnp.float32).astype(bf16)
            for b in range(B_PER):
                for hh in range(HQ_PER):
                    q = q2[b * SQ:(b + 1) * SQ, hh * DH:(hh + 1) * DH]
                    k = k_ref[t, b, hh]
                    s = lax.dot_general(
                        q, k, (((1,), (1,)), ((), ())),
                        preferred_element_type=jnp.float32) * 0.125
                    m = jnp.max(s, axis=-1, keepdims=True)
                    w = jnp.exp(s - m)
                    w = w / jnp.sum(w, axis=-1, keepdims=True)
                    ctx = lax.dot(w.astype(bf16), v_ref[t, b, hh],
                                  preferred_element_type=jnp.float32)
                    ctx_scratch[b * SQ:(b + 1) * SQ,
                                hh * DH:(hh + 1) * DH] = ctx.astype(bf16)
            part = lax.dot(ctx_scratch[...], wo_val,
                           preferred_element_type=jnp.float32)
            if is_first:
                out_ref[...] = part
            else:
                out_ref[...] += part

        contrib(0, wq_ref[...], wo_ref[...], is_first=True)
        for s in range(N_DEV - 1):
            rds[s][0].wait_recv()
            rds[s][1].wait_recv()
            contrib(s + 1, gwq[s], gwo[s])
        for rd_wq, rd_wo in rds:
            rd_wq.wait_send()
            rd_wo.wait_send()

    out2d = pl.pallas_call(
        body,
        out_shape=jax.ShapeDtypeStruct((B_PER * SQ, D_MODEL), jnp.float32),
        in_specs=[pl.BlockSpec(memory_space=pltpu.VMEM)] * 5,
        out_specs=pl.BlockSpec(memory_space=pltpu.VMEM),
        scratch_shapes=[
            pltpu.VMEM((N_DEV - 1, D_MODEL, QKD), bf16),
            pltpu.VMEM((N_DEV - 1, QKD, D_MODEL), bf16),
            pltpu.VMEM((B_PER * SQ, QKD), bf16),
            pltpu.SemaphoreType.DMA((N_DEV - 1,)),
            pltpu.SemaphoreType.DMA((N_DEV - 1,)),
            pltpu.SemaphoreType.DMA((N_DEV - 1,)),
            pltpu.SemaphoreType.DMA((N_DEV - 1,)),
        ],
        compiler_params=pltpu.CompilerParams(collective_id=0),
    )(x2d, wq, wo, k_arr, v_arr)

    return out2d.reshape(B_PER, SQ, D_MODEL)


# device time: 41601 ns/iter; 1.1394x vs baseline; 1.1394x over previous
import jax
import jax.numpy as jnp
from jax import lax
from jax.experimental import pallas as pl
from jax.experimental.pallas import tpu as pltpu

N_DEV = 8
B_PER = 2
SQ = 128
SKV = 128
HQ_PER = 4
DH = 64
D_MODEL = 512
QKD = HQ_PER * DH

_MASKS = (1, 2, 4, 3, 6, 5, 7)
_SLOT = {m: s for s, m in enumerate(_MASKS)}


def _code(p):
    return p ^ ((p >> 1) & 1)


def kernel(x, Wq, K_ext, V_ext, Wo):
    my = lax.axis_index("i")
    bf16 = jnp.bfloat16

    x2d = x.reshape(B_PER * SQ, D_MODEL).astype(bf16)
    wq = Wq.astype(bf16)
    wo = Wo.astype(bf16)

    def prep(ext):
        eb = lax.dynamic_slice_in_dim(ext, B_PER * my, B_PER, axis=0)
        eb = eb.astype(bf16).reshape(B_PER, SKV, N_DEV, HQ_PER, DH)
        masks = jnp.asarray((0,) + _MASKS)
        idx = _code(_code(my) ^ masks)
        eb = jnp.take(eb, idx, axis=2)
        return jnp.transpose(eb, (2, 0, 3, 1, 4))

    k_arr = prep(K_ext)
    v_arr = prep(V_ext)

    def body(x_ref, wq_ref, wo_ref, k_ref, v_ref, out_ref,
             gwq, gwo, ctx_scratch,
             send_wq, recv_wq, send_wo, recv_wo):
        my_code = _code(lax.axis_index("i"))

        barrier = pltpu.get_barrier_semaphore()
        for m in (1, 2, 4, 7):
            peer = _code(my_code ^ m)
            pl.semaphore_signal(barrier, inc=1, device_id=(peer,),
                                device_id_type=pl.DeviceIdType.MESH)
        pl.semaphore_wait(barrier, 4)

        send_ctr = iter(range(N_DEV - 1))

        def push(src_wq, src_wo, dst_slot, tgt_mask):
            s = next(send_ctr)
            peer = _code(my_code ^ tgt_mask)
            rd_wq = pltpu.make_async_remote_copy(
                src_ref=src_wq, dst_ref=gwq.at[dst_slot],
                send_sem=send_wq.at[s], recv_sem=recv_wq.at[dst_slot],
                device_id=(peer,), device_id_type=pl.DeviceIdType.MESH)
            rd_wo = pltpu.make_async_remote_copy(
                src_ref=src_wo, dst_ref=gwo.at[dst_slot],
                send_sem=send_wo.at[s], recv_sem=recv_wo.at[dst_slot],
                device_id=(peer,), device_id_type=pl.DeviceIdType.MESH)
            rd_wq.start()
            rd_wo.start()
            return rd_wq, rd_wo

        x_val = x_ref[...]

        def contrib(t, wq_val, wo_val, is_first=False):
            q2 = lax.dot(x_val, wq_val,
                         preferred_element_type=jnp.float32).astype(bf16)
            for b in range(B_PER):
                for hh in range(HQ_PER):
                    q = q2[b * SQ:(b + 1) * SQ, hh * DH:(hh + 1) * DH]
                    k = k_ref[t, b, hh]
                    s = lax.dot_general(
                        q, k, (((1,), (1,)), ((), ())),
                        preferred_element_type=jnp.float32) * 0.125
                    m = jnp.max(s, axis=-1, keepdims=True)
                    w = jnp.exp(s - m)
                    w = w / jnp.sum(w, axis=-1, keepdims=True)
                    ctx = lax.dot(w.astype(bf16), v_ref[t, b, hh],
                                  preferred_element_type=jnp.float32)
                    ctx_scratch[b * SQ:(b + 1) * SQ,
                                hh * DH:(hh + 1) * DH] = ctx.astype(bf16)
            part = lax.dot(ctx_scratch[...], wo_val,
                           preferred_element_type=jnp.float32)
            if is_first:
                out_ref[...] = part
            else:
                out_ref[...] += part

        all_rds = []

        for m in (1, 2, 4, 7):
            all_rds.append(push(wq_ref, wo_ref, _SLOT[m], m))

        contrib(0, wq_ref[...], wo_ref[...], is_first=True)

        relay_plan = {1: (4, 5), 2: (1, 3), 4: (2, 6)}
        for t, m in enumerate(_MASKS[:3], start=1):
            slot = _SLOT[m]
            all_rds[t - 1][0].wait_recv()
            all_rds[t - 1][1].wait_recv()
            fwd_mask, as_mask = relay_plan[m]
            all_rds.append(
                push(gwq.at[slot], gwo.at[slot], _SLOT[as_mask], fwd_mask))
            contrib(t, gwq[slot], gwo[slot])

        relay_rds = all_rds[4:7]
        for t, m in enumerate(_MASKS[3:6], start=4):
            slot = _SLOT[m]
            rd_wq = pltpu.make_async_remote_copy(
                src_ref=wq_ref, dst_ref=gwq.at[slot],
                send_sem=send_wq.at[0], recv_sem=recv_wq.at[slot],
                device_id=(my_code,), device_id_type=pl.DeviceIdType.MESH)
            rd_wo = pltpu.make_async_remote_copy(
                src_ref=wo_ref, dst_ref=gwo.at[slot],
                send_sem=send_wo.at[0], recv_sem=recv_wo.at[slot],
                device_id=(my_code,), device_id_type=pl.DeviceIdType.MESH)
            rd_wq.wait_recv()
            rd_wo.wait_recv()
            contrib(t, gwq[slot], gwo[slot])

        slot7 = _SLOT[7]
        all_rds[3][0].wait_recv()
        all_rds[3][1].wait_recv()
        contrib(7, gwq[slot7], gwo[slot7])

        for rd_wq, rd_wo in all_rds:
            rd_wq.wait_send()
            rd_wo.wait_send()

    out2d = pl.pallas_call(
        body,
        out_shape=jax.ShapeDtypeStruct((B_PER * SQ, D_MODEL), jnp.float32),
        in_specs=[pl.BlockSpec(memory_space=pltpu.VMEM)] * 5,
        out_specs=pl.BlockSpec(memory_space=pltpu.VMEM),
        scratch_shapes=[
            pltpu.VMEM((N_DEV - 1, D_MODEL, QKD), bf16),
            pltpu.VMEM((N_DEV - 1, QKD, D_MODEL), bf16),
            pltpu.VMEM((B_PER * SQ, QKD), bf16),
            pltpu.SemaphoreType.DMA((N_DEV - 1,)),
            pltpu.SemaphoreType.DMA((N_DEV - 1,)),
            pltpu.SemaphoreType.DMA((N_DEV - 1,)),
            pltpu.SemaphoreType.DMA((N_DEV - 1,)),
        ],
        compiler_params=pltpu.CompilerParams(collective_id=0),
    )(x2d, wq, wo, k_arr, v_arr)

    return out2d.reshape(B_PER, SQ, D_MODEL)


# device time: 33113 ns/iter; 1.4315x vs baseline; 1.2563x over previous
import jax
import jax.numpy as jnp
from jax import lax
from jax.experimental import pallas as pl
from jax.experimental.pallas import tpu as pltpu

N_DEV = 8
B_PER = 2
SQ = 128
SKV = 128
HQ_PER = 4
DH = 64
D_MODEL = 512
QKD = HQ_PER * DH

_MASKS = (1, 2, 4, 3, 6, 5, 7)
_SLOT = {m: s for s, m in enumerate(_MASKS)}
_RELAY = {1: (4, 5), 2: (1, 3), 4: (2, 6)}


def _code(p):
    return p ^ ((p >> 1) & 1)


def kernel(x, Wq, K_ext, V_ext, Wo):
    my = lax.axis_index("i")
    bf16 = jnp.bfloat16

    x2d = x.reshape(B_PER * SQ, D_MODEL).astype(bf16)
    wq = Wq.astype(bf16)
    wo = Wo.astype(bf16)

    def prep(ext):
        eb = lax.dynamic_slice_in_dim(ext, B_PER * my, B_PER, axis=0)
        eb = eb.astype(bf16).reshape(B_PER, SKV, N_DEV, HQ_PER, DH)
        return jnp.transpose(eb, (2, 0, 3, 1, 4))

    k_arr = prep(K_ext)
    v_arr = prep(V_ext)

    def body(x_ref, wq_ref, wo_ref, k_ref, v_ref, out_ref,
             gwq, gwo, ctx_scratch,
             send_wq, recv_wq, send_wo, recv_wo):
        my_code = _code(lax.axis_index("i"))

        barrier = pltpu.get_barrier_semaphore()
        for m in (1, 2, 4, 7):
            peer = _code(my_code ^ m)
            pl.semaphore_signal(barrier, inc=1, device_id=(peer,),
                                device_id_type=pl.DeviceIdType.MESH)
        pl.semaphore_wait(barrier, 4)

        ctr_wq = iter(range(N_DEV - 1))
        ctr_wo = iter(range(N_DEV - 1))

        def push(which, src, dst_slot, tgt_mask):
            peer = _code(my_code ^ tgt_mask)
            buf, s_sems, r_sems, ctr = (
                (gwq, send_wq, recv_wq, ctr_wq) if which == "wq"
                else (gwo, send_wo, recv_wo, ctr_wo))
            rd = pltpu.make_async_remote_copy(
                src_ref=src, dst_ref=buf.at[dst_slot],
                send_sem=s_sems.at[next(ctr)], recv_sem=r_sems.at[dst_slot],
                device_id=(peer,), device_id_type=pl.DeviceIdType.MESH)
            rd.start()
            return rd

        x_val = x_ref[...]

        def attn_phase(origin, wq_val):
            q2 = lax.dot(x_val, wq_val,
                         preferred_element_type=jnp.float32).astype(bf16)
            for b in range(B_PER):
                for hh in range(HQ_PER):
                    q = q2[b * SQ:(b + 1) * SQ, hh * DH:(hh + 1) * DH]
                    k = k_ref[origin, b, hh]
                    s = lax.dot_general(
                        q, k, (((1,), (1,)), ((), ())),
                        preferred_element_type=jnp.float32) * 0.125
                    m = jnp.max(s, axis=-1, keepdims=True)
                    w = jnp.exp(s - m)
                    w = w / jnp.sum(w, axis=-1, keepdims=True)
                    ctx = lax.dot(w.astype(bf16), v_ref[origin, b, hh],
                                  preferred_element_type=jnp.float32)
                    ctx_scratch[b * SQ:(b + 1) * SQ,
                                hh * DH:(hh + 1) * DH] = ctx.astype(bf16)

        def out_phase(wo_val, is_first=False):
            part = lax.dot(ctx_scratch[...], wo_val,
                           preferred_element_type=jnp.float32)
            if is_first:
                out_ref[...] = part
            else:
                out_ref[...] += part

        def recv_only(buf, r_sems, slot):
            return pltpu.make_async_remote_copy(
                src_ref=buf.at[slot], dst_ref=buf.at[slot],
                send_sem=send_wq.at[0], recv_sem=r_sems.at[slot],
                device_id=(my_code,), device_id_type=pl.DeviceIdType.MESH)

        import os as _os
        _exp = _os.environ.get("SCB_EXP", "")
        if _exp == "compute":
            attn_phase(_code(my_code), wq_ref[...])
            out_phase(wo_ref[...], is_first=True)
            for m in _MASKS:
                attn_phase(_code(my_code ^ m), wq_ref[...])
                out_phase(wo_ref[...])
            return
        if _exp == "comm":
            sends = []
            for m in (1, 2, 4, 7):
                sends.append(push("wq", wq_ref, _SLOT[m], m))
            for m in (1, 2, 4, 7):
                sends.append(push("wo", wo_ref, _SLOT[m], m))
            a_wq, a_wo = sends[0:4], sends[4:8]
            for i, m in enumerate(_MASKS[:3]):
                slot = _SLOT[m]
                fwd, as_m = _RELAY[m]
                a_wq[i].wait_recv()
                sends.append(push("wq", gwq.at[slot], _SLOT[as_m], fwd))
                a_wo[i].wait_recv()
                sends.append(push("wo", gwo.at[slot], _SLOT[as_m], fwd))
            for m in _MASKS[3:6]:
                slot = _SLOT[m]
                recv_only(gwq, recv_wq, slot).wait_recv()
                recv_only(gwo, recv_wo, slot).wait_recv()
            a_wq[3].wait_recv()
            a_wo[3].wait_recv()
            for rd in sends:
                rd.wait_send()
            out_ref[...] = jnp.zeros_like(out_ref)
            return

        sends = []

        for m in (1, 2, 4, 7):
            sends.append(push("wq", wq_ref, _SLOT[m], m))
        for m in (1, 2, 4, 7):
            sends.append(push("wo", wo_ref, _SLOT[m], m))
        a_wq = sends[0:4]
        a_wo = sends[4:8]

        attn_phase(_code(my_code), wq_ref[...])
        out_phase(wo_ref[...], is_first=True)

        for i, m in enumerate(_MASKS[:3]):
            slot = _SLOT[m]
            fwd, as_m = _RELAY[m]
            a_wq[i].wait_recv()
            sends.append(push("wq", gwq.at[slot], _SLOT[as_m], fwd))
            attn_phase(_code(my_code ^ m), gwq[slot])
            a_wo[i].wait_recv()
            sends.append(push("wo", gwo.at[slot], _SLOT[as_m], fwd))
            out_phase(gwo[slot])

        for m in _MASKS[3:6]:
            slot = _SLOT[m]
            recv_only(gwq, recv_wq, slot).wait_recv()
            attn_phase(_code(my_code ^ m), gwq[slot])
            recv_only(gwo, recv_wo, slot).wait_recv()
            out_phase(gwo[slot])

        slot7 = _SLOT[7]
        a_wq[3].wait_recv()
        attn_phase(_code(my_code ^ 7), gwq[slot7])
        a_wo[3].wait_recv()
        out_phase(gwo[slot7])

        for rd in sends:
            rd.wait_send()

    out2d = pl.pallas_call(
        body,
        out_shape=jax.ShapeDtypeStruct((B_PER * SQ, D_MODEL), jnp.float32),
        in_specs=[pl.BlockSpec(memory_space=pltpu.VMEM)] * 5,
        out_specs=pl.BlockSpec(memory_space=pltpu.VMEM),
        scratch_shapes=[
            pltpu.VMEM((N_DEV - 1, D_MODEL, QKD), bf16),
            pltpu.VMEM((N_DEV - 1, QKD, D_MODEL), bf16),
            pltpu.VMEM((B_PER * SQ, QKD), bf16),
            pltpu.SemaphoreType.DMA((N_DEV - 1,)),
            pltpu.SemaphoreType.DMA((N_DEV - 1,)),
            pltpu.SemaphoreType.DMA((N_DEV - 1,)),
            pltpu.SemaphoreType.DMA((N_DEV - 1,)),
        ],
        compiler_params=pltpu.CompilerParams(collective_id=0),
    )(x2d, wq, wo, k_arr, v_arr)

    return out2d.reshape(B_PER, SQ, D_MODEL)
